# baseline (speedup 1.0000x reference)
import functools

import jax
import jax.numpy as jnp
from jax import lax
from jax.experimental import pallas as pl
from jax.experimental.pallas import tpu as pltpu
from jax.experimental.pallas import tpu_sc as plsc

NC = 2
NS = 16
NW = NC * NS
CH = 80


def _sc_mesh():
  return plsc.VectorSubcoreMesh(core_axis_name="c", subcore_axis_name="s")


def _make_deg_kernel(n_nodes, n_edges, ch):
  ept = n_edges // NW
  nch = ept // ch

  @functools.partial(
      pl.kernel,
      out_type=jax.ShapeDtypeStruct((NC, n_nodes), jnp.float32),
      mesh=_sc_mesh(),
      scratch_types=[
          pltpu.VMEM_SHARED((n_nodes,), jnp.float32),
          pltpu.VMEM((nch, ch), jnp.int32),
          pltpu.VMEM((ch,), jnp.float32),
      ],
      compiler_params=pltpu.CompilerParams(use_tc_tiling_on_sc=False,
                                           disable_bounds_checks=True),
  )
  def deg_kernel(dst_hbm, ones_hbm, zeros_hbm, out_hbm, acc, didx, ones_v):
    c = lax.axis_index("c")
    s = lax.axis_index("s")
    wid = s * NC + c

    @pl.when(s == 0)
    def _zero():
      pltpu.sync_copy(zeros_hbm, acc)

    pltpu.sync_copy(ones_hbm, ones_v)
    pltpu.sync_copy(dst_hbm.at[pl.ds(wid * nch, nch)], didx)
    plsc.subcore_barrier()

    def body(i, carry):
      pltpu.sync_copy(ones_v, acc.at[didx.at[i]], add=True)
      return carry

    lax.fori_loop(0, nch, body, 0)
    plsc.subcore_barrier()

    @pl.when(s == 0)
    def _out():
      pltpu.sync_copy(acc, out_hbm.at[c])

  return deg_kernel


def _make_agg_kernel(n_nodes, n_edges, d, ch, stage_src=False):
  ept = n_edges // NW
  nch = ept // ch
  rpt = n_nodes // NS

  scratch = [
      pltpu.VMEM_SHARED((n_nodes, d), jnp.float32),
      pltpu.VMEM((nch, ch), jnp.int32),
      pltpu.VMEM((nch, ch), jnp.int32),
      pltpu.VMEM((ch, d), jnp.float32),
      pltpu.VMEM((ch, d), jnp.float32),
      pltpu.SemaphoreType.DMA,
      pltpu.SemaphoreType.DMA,
      pltpu.SemaphoreType.DMA,
      pltpu.SemaphoreType.DMA,
  ]
  if stage_src:
    scratch.append(pltpu.VMEM_SHARED((n_nodes, d), jnp.float32))

  @functools.partial(
      pl.kernel,
      out_type=jax.ShapeDtypeStruct((NC, n_nodes, d), jnp.float32),
      mesh=_sc_mesh(),
      scratch_types=scratch,
      compiler_params=pltpu.CompilerParams(use_tc_tiling_on_sc=False,
                                           disable_bounds_checks=True),
  )
  def agg_kernel(y_hbm, src_hbm, dst_hbm, zeros_hbm, out_hbm,
                 acc, sidx, didx, rows0, rows1, sem0, sem1, ssem0, ssem1,
                 *maybe_ysp):
    c = lax.axis_index("c")
    s = lax.axis_index("s")
    wid = s * NC + c

    pltpu.sync_copy(zeros_hbm, acc.at[pl.ds(s * rpt, rpt)])
    pltpu.sync_copy(src_hbm.at[pl.ds(wid * nch, nch)], sidx)
    pltpu.sync_copy(dst_hbm.at[pl.ds(wid * nch, nch)], didx)
    if stage_src:
      pltpu.sync_copy(y_hbm.at[pl.ds(s * rpt, rpt)],
                      maybe_ysp[0].at[pl.ds(s * rpt, rpt)])
      y_hbm = maybe_ysp[0]
    plsc.subcore_barrier()

    pltpu.async_copy(y_hbm.at[sidx.at[0]], rows0, sem0)

    def pair(j, carry):
      i0 = 2 * j
      i1 = i0 + 1
      pltpu.make_async_copy(y_hbm.at[sidx.at[i0]], rows0, sem0).wait()
      pltpu.async_copy(y_hbm.at[sidx.at[i1]], rows1, sem1)
      pltpu.async_copy(rows0, acc.at[didx.at[i0]], ssem0, add=True)
      pltpu.make_async_copy(y_hbm.at[sidx.at[i1]], rows1, sem1).wait()
      pltpu.async_copy(rows1, acc.at[didx.at[i1]], ssem1, add=True)
      pltpu.make_async_copy(rows0, acc.at[didx.at[i0]], ssem0).wait()
      pltpu.async_copy(y_hbm.at[sidx.at[i0 + 2]], rows0, sem0)
      pltpu.make_async_copy(rows1, acc.at[didx.at[i1]], ssem1).wait()
      return carry

    lax.fori_loop(0, (nch - 1) // 2, pair, 0)
    i_last = nch - 1
    pltpu.make_async_copy(y_hbm.at[sidx.at[i_last]], rows0, sem0).wait()
    pltpu.sync_copy(rows0, acc.at[didx.at[i_last]], add=True)
    plsc.subcore_barrier()

    pltpu.sync_copy(acc.at[pl.ds(s * rpt, rpt)],
                    out_hbm.at[c, pl.ds(s * rpt, rpt)])

  return agg_kernel


def _tc_prep(degp_ref, x_ref, y_ref):
  deg = degp_ref[0] + degp_ref[1] + 1.0
  y_ref[...] = lax.rsqrt(deg) * x_ref[...]


def _tc_mid(degp_ref, p_ref, y_ref, w1_ref, b1_ref, w2_ref, y2_ref):
  deg = degp_ref[0] + degp_ref[1] + 1.0
  dis = lax.rsqrt(deg)
  agg1 = dis * (p_ref[0] + p_ref[1] + y_ref[...])
  h1 = jnp.maximum(
      jnp.dot(agg1, w1_ref[...], preferred_element_type=jnp.float32)
      + b1_ref[...], 0.0)
  y2_ref[...] = dis * jnp.dot(h1, w2_ref[...],
                              preferred_element_type=jnp.float32)


def _tc_out(degp_ref, p_ref, y2_ref, b2_ref, o_ref):
  deg = degp_ref[0] + degp_ref[1] + 1.0
  dis = lax.rsqrt(deg)
  agg2 = dis * (p_ref[0] + p_ref[1] + y2_ref[...]) + b2_ref[...]
  m = jnp.max(agg2, axis=-1, keepdims=True)
  e = jnp.exp(agg2 - m)
  o_ref[...] = e / jnp.sum(e, axis=-1, keepdims=True)


def kernel(x, edge_index, W1, b1, W2, b2):
  n, d_in = x.shape
  e = edge_index.shape[1]
  d_hid = W1.shape[1]
  n_cls = W2.shape[1]

  src = edge_index[0].astype(jnp.int32).reshape(e // CH, CH)
  dst = edge_index[1].astype(jnp.int32).reshape(e // CH, CH)

  ones_ch = jnp.ones((CH,), jnp.float32)
  zeros_n = jnp.zeros((n,), jnp.float32)
  zeros_1 = jnp.zeros((n // NS, d_in), jnp.float32)
  zeros_2 = jnp.zeros((n // NS, n_cls), jnp.float32)

  degp = _make_deg_kernel(n, e, CH)(dst, ones_ch, zeros_n)
  degp3 = degp.reshape(NC, n, 1)

  blk = 2000
  grid = (n // blk,)

  y = pl.pallas_call(
      _tc_prep,
      grid=grid,
      in_specs=[
          pl.BlockSpec((NC, blk, 1), lambda j: (0, j, 0)),
          pl.BlockSpec((blk, d_in), lambda j: (j, 0)),
      ],
      out_specs=pl.BlockSpec((blk, d_in), lambda j: (j, 0)),
      out_shape=jax.ShapeDtypeStruct((n, d_in), jnp.float32),
  )(degp3, x)

  part1 = _make_agg_kernel(n, e, d_in, CH)(y, src, dst, zeros_1)

  y2 = pl.pallas_call(
      _tc_mid,
      grid=grid,
      in_specs=[
          pl.BlockSpec((NC, blk, 1), lambda j: (0, j, 0)),
          pl.BlockSpec((NC, blk, d_in), lambda j: (0, j, 0)),
          pl.BlockSpec((blk, d_in), lambda j: (j, 0)),
          pl.BlockSpec((d_in, d_hid), lambda j: (0, 0)),
          pl.BlockSpec((1, d_hid), lambda j: (0, 0)),
          pl.BlockSpec((d_hid, n_cls), lambda j: (0, 0)),
      ],
      out_specs=pl.BlockSpec((blk, n_cls), lambda j: (j, 0)),
      out_shape=jax.ShapeDtypeStruct((n, n_cls), jnp.float32),
  )(degp3, part1, y, W1, b1.reshape(1, d_hid), W2)

  part2 = _make_agg_kernel(n, e, n_cls, CH, stage_src=True)(
      y2, src, dst, zeros_2)

  out = pl.pallas_call(
      _tc_out,
      grid=grid,
      in_specs=[
          pl.BlockSpec((NC, blk, 1), lambda j: (0, j, 0)),
          pl.BlockSpec((NC, blk, n_cls), lambda j: (0, j, 0)),
          pl.BlockSpec((blk, n_cls), lambda j: (j, 0)),
          pl.BlockSpec((1, n_cls), lambda j: (0, 0)),
      ],
      out_specs=pl.BlockSpec((blk, n_cls), lambda j: (j, 0)),
      out_shape=jax.ShapeDtypeStruct((n, n_cls), jnp.float32),
  )(degp3, part2, y2, b2.reshape(1, n_cls))

  return out


# speedup vs baseline: 1.0396x; 1.0396x over previous
import functools

import jax
import jax.numpy as jnp
from jax import lax
from jax.experimental import pallas as pl
from jax.experimental.pallas import tpu as pltpu
from jax.experimental.pallas import tpu_sc as plsc

NC = 2
NS = 16
NW = NC * NS
CH = 80


def _sc_mesh():
  return plsc.VectorSubcoreMesh(core_axis_name="c", subcore_axis_name="s")


def _make_deg_kernel(n_nodes, n_edges, ch):
  ept = n_edges // NW
  nch = ept // ch

  @functools.partial(
      pl.kernel,
      out_type=jax.ShapeDtypeStruct((NC, n_nodes), jnp.float32),
      mesh=_sc_mesh(),
      scratch_types=[
          pltpu.VMEM_SHARED((n_nodes,), jnp.float32),
          pltpu.VMEM((nch, ch), jnp.int32),
          pltpu.VMEM((ch,), jnp.float32),
      ],
      compiler_params=pltpu.CompilerParams(use_tc_tiling_on_sc=False,
                                           disable_bounds_checks=True),
  )
  def deg_kernel(ei_hbm, ones_hbm, zeros_hbm, out_hbm, acc, didx, ones_v):
    c = lax.axis_index("c")
    s = lax.axis_index("s")
    wid = s * NC + c

    @pl.when(s == 0)
    def _zero():
      pltpu.sync_copy(zeros_hbm, acc)

    pltpu.sync_copy(ones_hbm, ones_v)
    pltpu.sync_copy(ei_hbm.at[1, pl.ds(wid * nch, nch)], didx)
    plsc.subcore_barrier()

    def body(i, carry):
      pltpu.sync_copy(ones_v, acc.at[didx.at[i]], add=True)
      return carry

    lax.fori_loop(0, nch, body, 0)
    plsc.subcore_barrier()

    @pl.when(s == 0)
    def _out():
      pltpu.sync_copy(acc, out_hbm.at[c])

  return deg_kernel


def _make_agg_kernel(n_nodes, n_edges, d, ch, stage_src=False):
  ept = n_edges // NW
  nch = ept // ch
  rpt = n_nodes // NS

  scratch = [
      pltpu.VMEM_SHARED((n_nodes, d), jnp.float32),
      pltpu.VMEM((nch, ch), jnp.int32),
      pltpu.VMEM((nch, ch), jnp.int32),
      pltpu.VMEM((ch, d), jnp.float32),
      pltpu.VMEM((ch, d), jnp.float32),
      pltpu.SemaphoreType.DMA,
      pltpu.SemaphoreType.DMA,
      pltpu.SemaphoreType.DMA,
      pltpu.SemaphoreType.DMA,
  ]
  if stage_src:
    scratch.append(pltpu.VMEM_SHARED((n_nodes, d), jnp.float32))

  @functools.partial(
      pl.kernel,
      out_type=jax.ShapeDtypeStruct((NC, n_nodes, d), jnp.float32),
      mesh=_sc_mesh(),
      scratch_types=scratch,
      compiler_params=pltpu.CompilerParams(use_tc_tiling_on_sc=False,
                                           disable_bounds_checks=True),
  )
  def agg_kernel(y_hbm, ei_hbm, zeros_hbm, out_hbm,
                 acc, sidx, didx, rows0, rows1, sem0, sem1, ssem0, ssem1,
                 *maybe_ysp):
    c = lax.axis_index("c")
    s = lax.axis_index("s")
    wid = s * NC + c

    pltpu.sync_copy(zeros_hbm, acc.at[pl.ds(s * rpt, rpt)])
    pltpu.sync_copy(ei_hbm.at[0, pl.ds(wid * nch, nch)], sidx)
    pltpu.sync_copy(ei_hbm.at[1, pl.ds(wid * nch, nch)], didx)
    if stage_src:
      pltpu.sync_copy(y_hbm.at[pl.ds(s * rpt, rpt)],
                      maybe_ysp[0].at[pl.ds(s * rpt, rpt)])
      y_hbm = maybe_ysp[0]
    plsc.subcore_barrier()

    pltpu.async_copy(y_hbm.at[sidx.at[0]], rows0, sem0)

    def pair(j, carry):
      i0 = 2 * j
      i1 = i0 + 1
      pltpu.make_async_copy(y_hbm.at[sidx.at[i0]], rows0, sem0).wait()
      pltpu.async_copy(y_hbm.at[sidx.at[i1]], rows1, sem1)
      pltpu.async_copy(rows0, acc.at[didx.at[i0]], ssem0, add=True)
      pltpu.make_async_copy(y_hbm.at[sidx.at[i1]], rows1, sem1).wait()
      pltpu.async_copy(rows1, acc.at[didx.at[i1]], ssem1, add=True)
      pltpu.make_async_copy(rows0, acc.at[didx.at[i0]], ssem0).wait()
      pltpu.async_copy(y_hbm.at[sidx.at[i0 + 2]], rows0, sem0)
      pltpu.make_async_copy(rows1, acc.at[didx.at[i1]], ssem1).wait()
      return carry

    lax.fori_loop(0, (nch - 1) // 2, pair, 0)
    i_last = nch - 1
    pltpu.make_async_copy(y_hbm.at[sidx.at[i_last]], rows0, sem0).wait()
    pltpu.sync_copy(rows0, acc.at[didx.at[i_last]], add=True)
    plsc.subcore_barrier()

    pltpu.sync_copy(acc.at[pl.ds(s * rpt, rpt)],
                    out_hbm.at[c, pl.ds(s * rpt, rpt)])

  return agg_kernel


def _tc_prep(degp_ref, x_ref, y_ref):
  deg = degp_ref[0] + degp_ref[1] + 1.0
  y_ref[...] = lax.rsqrt(deg) * x_ref[...]


def _tc_mid(degp_ref, p_ref, y_ref, w1_ref, b1_ref, w2_ref, y2_ref):
  deg = degp_ref[0] + degp_ref[1] + 1.0
  dis = lax.rsqrt(deg)
  agg1 = dis * (p_ref[0] + p_ref[1] + y_ref[...])
  h1 = jnp.maximum(
      jnp.dot(agg1, w1_ref[...], preferred_element_type=jnp.float32)
      + b1_ref[...], 0.0)
  y2_ref[...] = dis * jnp.dot(h1, w2_ref[...],
                              preferred_element_type=jnp.float32)


def _tc_out(degp_ref, p_ref, y2_ref, b2_ref, o_ref):
  deg = degp_ref[0] + degp_ref[1] + 1.0
  dis = lax.rsqrt(deg)
  agg2 = dis * (p_ref[0] + p_ref[1] + y2_ref[...]) + b2_ref[...]
  m = jnp.max(agg2, axis=-1, keepdims=True)
  e = jnp.exp(agg2 - m)
  o_ref[...] = e / jnp.sum(e, axis=-1, keepdims=True)


def kernel(x, edge_index, W1, b1, W2, b2):
  n, d_in = x.shape
  e = edge_index.shape[1]
  d_hid = W1.shape[1]
  n_cls = W2.shape[1]

  ei3d = edge_index.astype(jnp.int32).reshape(2, e // CH, CH)

  ones_ch = jnp.ones((CH,), jnp.float32)
  zeros_n = jnp.zeros((n,), jnp.float32)
  zeros_1 = jnp.zeros((n // NS, d_in), jnp.float32)
  zeros_2 = jnp.zeros((n // NS, n_cls), jnp.float32)

  degp = _make_deg_kernel(n, e, CH)(ei3d, ones_ch, zeros_n)
  degp3 = degp.reshape(NC, n, 1)

  blk = 2000
  grid = (n // blk,)

  y = pl.pallas_call(
      _tc_prep,
      grid=grid,
      in_specs=[
          pl.BlockSpec((NC, blk, 1), lambda j: (0, j, 0)),
          pl.BlockSpec((blk, d_in), lambda j: (j, 0)),
      ],
      out_specs=pl.BlockSpec((blk, d_in), lambda j: (j, 0)),
      out_shape=jax.ShapeDtypeStruct((n, d_in), jnp.float32),
  )(degp3, x)

  part1 = _make_agg_kernel(n, e, d_in, CH)(y, ei3d, zeros_1)

  y2 = pl.pallas_call(
      _tc_mid,
      grid=grid,
      in_specs=[
          pl.BlockSpec((NC, blk, 1), lambda j: (0, j, 0)),
          pl.BlockSpec((NC, blk, d_in), lambda j: (0, j, 0)),
          pl.BlockSpec((blk, d_in), lambda j: (j, 0)),
          pl.BlockSpec((d_in, d_hid), lambda j: (0, 0)),
          pl.BlockSpec((1, d_hid), lambda j: (0, 0)),
          pl.BlockSpec((d_hid, n_cls), lambda j: (0, 0)),
      ],
      out_specs=pl.BlockSpec((blk, n_cls), lambda j: (j, 0)),
      out_shape=jax.ShapeDtypeStruct((n, n_cls), jnp.float32),
  )(degp3, part1, y, W1, b1.reshape(1, d_hid), W2)

  part2 = _make_agg_kernel(n, e, n_cls, CH, stage_src=True)(
      y2, ei3d, zeros_2)

  out = pl.pallas_call(
      _tc_out,
      grid=grid,
      in_specs=[
          pl.BlockSpec((NC, blk, 1), lambda j: (0, j, 0)),
          pl.BlockSpec((NC, blk, n_cls), lambda j: (0, j, 0)),
          pl.BlockSpec((blk, n_cls), lambda j: (j, 0)),
          pl.BlockSpec((1, n_cls), lambda j: (0, 0)),
      ],
      out_specs=pl.BlockSpec((blk, n_cls), lambda j: (j, 0)),
      out_shape=jax.ShapeDtypeStruct((n, n_cls), jnp.float32),
  )(degp3, part2, y2, b2.reshape(1, n_cls))

  return out


# speedup vs baseline: 1.0582x; 1.0179x over previous
import functools

import jax
import jax.numpy as jnp
from jax import lax
from jax.experimental import pallas as pl
from jax.experimental.pallas import tpu as pltpu
from jax.experimental.pallas import tpu_sc as plsc

NC = 2
NS = 16
NW = NC * NS
CH = 80


def _sc_mesh():
  return plsc.VectorSubcoreMesh(core_axis_name="c", subcore_axis_name="s")


def _make_deg_kernel(n_nodes, n_edges, ch):
  ept = n_edges // NW
  nch = ept // ch

  @functools.partial(
      pl.kernel,
      out_type=jax.ShapeDtypeStruct((NC, n_nodes), jnp.float32),
      mesh=_sc_mesh(),
      scratch_types=[
          pltpu.VMEM_SHARED((n_nodes,), jnp.float32),
          pltpu.VMEM((nch, ch), jnp.int32),
          pltpu.VMEM((ch,), jnp.float32),
          pltpu.SemaphoreType.DMA,
          pltpu.SemaphoreType.DMA,
      ],
      compiler_params=pltpu.CompilerParams(use_tc_tiling_on_sc=False,
                                           disable_bounds_checks=True),
  )
  def deg_kernel(ei_hbm, ones_hbm, zeros_hbm, out_hbm, acc, didx, ones_v,
                 sem0, sem1):
    c = lax.axis_index("c")
    s = lax.axis_index("s")
    wid = s * NC + c

    @pl.when(s == 0)
    def _zero():
      pltpu.sync_copy(zeros_hbm, acc)

    pltpu.sync_copy(ones_hbm, ones_v)
    pltpu.sync_copy(ei_hbm.at[1, pl.ds(wid * nch, nch)], didx)
    plsc.subcore_barrier()

    pltpu.async_copy(ones_v, acc.at[didx.at[0]], sem0, add=True)

    def pair(j, carry):
      i0 = 2 * j
      i1 = i0 + 1
      pltpu.async_copy(ones_v, acc.at[didx.at[i1]], sem1, add=True)
      pltpu.make_async_copy(ones_v, acc.at[didx.at[i0]], sem0).wait()
      pltpu.async_copy(ones_v, acc.at[didx.at[i0 + 2]], sem0, add=True)
      pltpu.make_async_copy(ones_v, acc.at[didx.at[i1]], sem1).wait()
      return carry

    lax.fori_loop(0, (nch - 1) // 2, pair, 0)
    pltpu.make_async_copy(ones_v, acc.at[didx.at[nch - 1]], sem0).wait()
    plsc.subcore_barrier()

    @pl.when(s == 0)
    def _out():
      pltpu.sync_copy(acc, out_hbm.at[c])

  return deg_kernel


def _make_agg_kernel(n_nodes, n_edges, d, ch, stage_src=False):
  ept = n_edges // NW
  nch = ept // ch
  rpt = n_nodes // NS

  scratch = [
      pltpu.VMEM_SHARED((n_nodes, d), jnp.float32),
      pltpu.VMEM((nch, ch), jnp.int32),
      pltpu.VMEM((nch, ch), jnp.int32),
      pltpu.VMEM((ch, d), jnp.float32),
      pltpu.VMEM((ch, d), jnp.float32),
      pltpu.SemaphoreType.DMA,
      pltpu.SemaphoreType.DMA,
      pltpu.SemaphoreType.DMA,
      pltpu.SemaphoreType.DMA,
  ]
  if stage_src:
    scratch.append(pltpu.VMEM_SHARED((n_nodes, d), jnp.float32))

  @functools.partial(
      pl.kernel,
      out_type=jax.ShapeDtypeStruct((NC, n_nodes, d), jnp.float32),
      mesh=_sc_mesh(),
      scratch_types=scratch,
      compiler_params=pltpu.CompilerParams(use_tc_tiling_on_sc=False,
                                           disable_bounds_checks=True),
  )
  def agg_kernel(y_hbm, ei_hbm, zeros_hbm, out_hbm,
                 acc, sidx, didx, rows0, rows1, sem0, sem1, ssem0, ssem1,
                 *maybe_ysp):
    c = lax.axis_index("c")
    s = lax.axis_index("s")
    wid = s * NC + c

    pltpu.sync_copy(zeros_hbm, acc.at[pl.ds(s * rpt, rpt)])
    pltpu.sync_copy(ei_hbm.at[0, pl.ds(wid * nch, nch)], sidx)
    pltpu.sync_copy(ei_hbm.at[1, pl.ds(wid * nch, nch)], didx)
    if stage_src:
      pltpu.sync_copy(y_hbm.at[pl.ds(s * rpt, rpt)],
                      maybe_ysp[0].at[pl.ds(s * rpt, rpt)])
      y_hbm = maybe_ysp[0]
    plsc.subcore_barrier()

    pltpu.async_copy(y_hbm.at[sidx.at[0]], rows0, sem0)

    def pair(j, carry):
      i0 = 2 * j
      i1 = i0 + 1
      pltpu.make_async_copy(y_hbm.at[sidx.at[i0]], rows0, sem0).wait()
      pltpu.async_copy(y_hbm.at[sidx.at[i1]], rows1, sem1)
      pltpu.async_copy(rows0, acc.at[didx.at[i0]], ssem0, add=True)
      pltpu.make_async_copy(y_hbm.at[sidx.at[i1]], rows1, sem1).wait()
      pltpu.async_copy(rows1, acc.at[didx.at[i1]], ssem1, add=True)
      pltpu.make_async_copy(rows0, acc.at[didx.at[i0]], ssem0).wait()
      pltpu.async_copy(y_hbm.at[sidx.at[i0 + 2]], rows0, sem0)
      pltpu.make_async_copy(rows1, acc.at[didx.at[i1]], ssem1).wait()
      return carry

    lax.fori_loop(0, (nch - 1) // 2, pair, 0)
    i_last = nch - 1
    pltpu.make_async_copy(y_hbm.at[sidx.at[i_last]], rows0, sem0).wait()
    pltpu.sync_copy(rows0, acc.at[didx.at[i_last]], add=True)
    plsc.subcore_barrier()

    pltpu.sync_copy(acc.at[pl.ds(s * rpt, rpt)],
                    out_hbm.at[c, pl.ds(s * rpt, rpt)])

  return agg_kernel


def _tc_prep(degp_ref, x_ref, y_ref):
  deg = degp_ref[0] + degp_ref[1] + 1.0
  y_ref[...] = lax.rsqrt(deg) * x_ref[...]


def _tc_mid(degp_ref, p_ref, y_ref, w1_ref, b1_ref, w2_ref, y2_ref):
  deg = degp_ref[0] + degp_ref[1] + 1.0
  dis = lax.rsqrt(deg)
  agg1 = dis * (p_ref[0] + p_ref[1] + y_ref[...])
  h1 = jnp.maximum(
      jnp.dot(agg1, w1_ref[...], preferred_element_type=jnp.float32)
      + b1_ref[...], 0.0)
  y2_ref[...] = dis * jnp.dot(h1, w2_ref[...],
                              preferred_element_type=jnp.float32)


def _tc_out(degp_ref, p_ref, y2_ref, b2_ref, o_ref):
  deg = degp_ref[0] + degp_ref[1] + 1.0
  dis = lax.rsqrt(deg)
  agg2 = dis * (p_ref[0] + p_ref[1] + y2_ref[...]) + b2_ref[...]
  m = jnp.max(agg2, axis=-1, keepdims=True)
  e = jnp.exp(agg2 - m)
  o_ref[...] = e / jnp.sum(e, axis=-1, keepdims=True)


def kernel(x, edge_index, W1, b1, W2, b2):
  n, d_in = x.shape
  e = edge_index.shape[1]
  d_hid = W1.shape[1]
  n_cls = W2.shape[1]

  ei3d = edge_index.astype(jnp.int32).reshape(2, e // CH, CH)

  ones_ch = jnp.ones((CH,), jnp.float32)
  zeros_n = jnp.zeros((n,), jnp.float32)
  zeros_1 = jnp.zeros((n // NS, d_in), jnp.float32)
  zeros_2 = jnp.zeros((n // NS, n_cls), jnp.float32)

  degp = _make_deg_kernel(n, e, CH)(ei3d, ones_ch, zeros_n)
  degp3 = degp.reshape(NC, n, 1)

  blk = 2000
  grid = (n // blk,)

  y = pl.pallas_call(
      _tc_prep,
      grid=grid,
      in_specs=[
          pl.BlockSpec((NC, blk, 1), lambda j: (0, j, 0)),
          pl.BlockSpec((blk, d_in), lambda j: (j, 0)),
      ],
      out_specs=pl.BlockSpec((blk, d_in), lambda j: (j, 0)),
      out_shape=jax.ShapeDtypeStruct((n, d_in), jnp.float32),
  )(degp3, x)

  part1 = _make_agg_kernel(n, e, d_in, CH)(y, ei3d, zeros_1)

  y2 = pl.pallas_call(
      _tc_mid,
      grid=grid,
      in_specs=[
          pl.BlockSpec((NC, blk, 1), lambda j: (0, j, 0)),
          pl.BlockSpec((NC, blk, d_in), lambda j: (0, j, 0)),
          pl.BlockSpec((blk, d_in), lambda j: (j, 0)),
          pl.BlockSpec((d_in, d_hid), lambda j: (0, 0)),
          pl.BlockSpec((1, d_hid), lambda j: (0, 0)),
          pl.BlockSpec((d_hid, n_cls), lambda j: (0, 0)),
      ],
      out_specs=pl.BlockSpec((blk, n_cls), lambda j: (j, 0)),
      out_shape=jax.ShapeDtypeStruct((n, n_cls), jnp.float32),
  )(degp3, part1, y, W1, b1.reshape(1, d_hid), W2)

  part2 = _make_agg_kernel(n, e, n_cls, CH, stage_src=True)(
      y2, ei3d, zeros_2)

  out = pl.pallas_call(
      _tc_out,
      grid=grid,
      in_specs=[
          pl.BlockSpec((NC, blk, 1), lambda j: (0, j, 0)),
          pl.BlockSpec((NC, blk, n_cls), lambda j: (0, j, 0)),
          pl.BlockSpec((blk, n_cls), lambda j: (j, 0)),
          pl.BlockSpec((1, n_cls), lambda j: (0, 0)),
      ],
      out_specs=pl.BlockSpec((blk, n_cls), lambda j: (j, 0)),
      out_shape=jax.ShapeDtypeStruct((n, n_cls), jnp.float32),
  )(degp3, part2, y2, b2.reshape(1, n_cls))

  return out


# speedup vs baseline: 1.0835x; 1.0239x over previous
import functools

import jax
import jax.numpy as jnp
from jax import lax
from jax.experimental import pallas as pl
from jax.experimental.pallas import tpu as pltpu
from jax.experimental.pallas import tpu_sc as plsc

NC = 2
NS = 16
NW = NC * NS
CH = 80
CH_S = 400


def _sc_mesh():
  return plsc.VectorSubcoreMesh(core_axis_name="c", subcore_axis_name="s")


def _make_deg_kernel(n_nodes, n_edges, ch):
  ept = n_edges // NW
  nch = ept // ch

  @functools.partial(
      pl.kernel,
      out_type=jax.ShapeDtypeStruct((NC, n_nodes), jnp.float32),
      mesh=_sc_mesh(),
      scratch_types=[
          pltpu.VMEM_SHARED((n_nodes,), jnp.float32),
          pltpu.VMEM((nch, ch), jnp.int32),
          pltpu.VMEM((ch,), jnp.float32),
          pltpu.SemaphoreType.DMA,
          pltpu.SemaphoreType.DMA,
      ],
      compiler_params=pltpu.CompilerParams(use_tc_tiling_on_sc=False,
                                           disable_bounds_checks=True),
  )
  def deg_kernel(ei_hbm, ones_hbm, zeros_hbm, out_hbm, acc, didx, ones_v,
                 sem0, sem1):
    c = lax.axis_index("c")
    s = lax.axis_index("s")
    wid = s * NC + c

    @pl.when(s == 0)
    def _zero():
      pltpu.sync_copy(zeros_hbm, acc)

    pltpu.sync_copy(ones_hbm, ones_v)
    pltpu.sync_copy(ei_hbm.at[1, pl.ds(wid * nch, nch)], didx)
    plsc.subcore_barrier()

    pltpu.async_copy(ones_v, acc.at[didx.at[0]], sem0, add=True)

    def pair(j, carry):
      i0 = 2 * j
      i1 = i0 + 1
      pltpu.async_copy(ones_v, acc.at[didx.at[i1]], sem1, add=True)
      pltpu.make_async_copy(ones_v, acc.at[didx.at[i0]], sem0).wait()
      pltpu.async_copy(ones_v, acc.at[didx.at[i0 + 2]], sem0, add=True)
      pltpu.make_async_copy(ones_v, acc.at[didx.at[i1]], sem1).wait()
      return carry

    lax.fori_loop(0, (nch - 1) // 2, pair, 0)
    pltpu.make_async_copy(ones_v, acc.at[didx.at[nch - 1]], sem0).wait()
    plsc.subcore_barrier()

    @pl.when(s == 0)
    def _out():
      pltpu.sync_copy(acc, out_hbm.at[c])

  return deg_kernel


def _make_agg_kernel(n_nodes, n_edges, d, ch, stage_src=False):
  ept = n_edges // NW
  nch = ept // ch
  rpt = n_nodes // NS

  scratch = [
      pltpu.VMEM_SHARED((n_nodes, d), jnp.float32),
      pltpu.VMEM((nch, ch), jnp.int32),
      pltpu.VMEM((nch, ch), jnp.int32),
      pltpu.VMEM((ch, d), jnp.float32),
      pltpu.VMEM((ch, d), jnp.float32),
      pltpu.SemaphoreType.DMA,
      pltpu.SemaphoreType.DMA,
      pltpu.SemaphoreType.DMA,
      pltpu.SemaphoreType.DMA,
  ]
  if stage_src:
    scratch.append(pltpu.VMEM_SHARED((n_nodes, d), jnp.float32))

  @functools.partial(
      pl.kernel,
      out_type=jax.ShapeDtypeStruct((NC, n_nodes, d), jnp.float32),
      mesh=_sc_mesh(),
      scratch_types=scratch,
      compiler_params=pltpu.CompilerParams(use_tc_tiling_on_sc=False,
                                           disable_bounds_checks=True),
  )
  def agg_kernel(y_hbm, ei_hbm, zeros_hbm, out_hbm,
                 acc, sidx, didx, rows0, rows1, sem0, sem1, ssem0, ssem1,
                 *maybe_ysp):
    c = lax.axis_index("c")
    s = lax.axis_index("s")
    wid = s * NC + c

    pltpu.sync_copy(zeros_hbm, acc.at[pl.ds(s * rpt, rpt)])
    pltpu.sync_copy(ei_hbm.at[0, pl.ds(wid * nch, nch)], sidx)
    pltpu.sync_copy(ei_hbm.at[1, pl.ds(wid * nch, nch)], didx)
    if stage_src:
      pltpu.sync_copy(y_hbm.at[pl.ds(s * rpt, rpt)],
                      maybe_ysp[0].at[pl.ds(s * rpt, rpt)])
      y_hbm = maybe_ysp[0]
    plsc.subcore_barrier()

    pltpu.async_copy(y_hbm.at[sidx.at[0]], rows0, sem0)

    def pair(j, carry):
      i0 = 2 * j
      i1 = i0 + 1
      pltpu.make_async_copy(y_hbm.at[sidx.at[i0]], rows0, sem0).wait()
      pltpu.async_copy(y_hbm.at[sidx.at[i1]], rows1, sem1)
      pltpu.async_copy(rows0, acc.at[didx.at[i0]], ssem0, add=True)
      pltpu.make_async_copy(y_hbm.at[sidx.at[i1]], rows1, sem1).wait()
      pltpu.async_copy(rows1, acc.at[didx.at[i1]], ssem1, add=True)
      pltpu.make_async_copy(rows0, acc.at[didx.at[i0]], ssem0).wait()
      pltpu.async_copy(y_hbm.at[sidx.at[i0 + 2]], rows0, sem0)
      pltpu.make_async_copy(rows1, acc.at[didx.at[i1]], ssem1).wait()
      return carry

    lax.fori_loop(0, (nch - 1) // 2, pair, 0)
    i_last = nch - 1
    pltpu.make_async_copy(y_hbm.at[sidx.at[i_last]], rows0, sem0).wait()
    pltpu.sync_copy(rows0, acc.at[didx.at[i_last]], add=True)
    plsc.subcore_barrier()

    pltpu.sync_copy(acc.at[pl.ds(s * rpt, rpt)],
                    out_hbm.at[c, pl.ds(s * rpt, rpt)])

  return agg_kernel


def _tc_prep(degp_ref, x_ref, y_ref):
  deg = degp_ref[0] + degp_ref[1] + 1.0
  y_ref[...] = lax.rsqrt(deg) * x_ref[...]


def _tc_mid(degp_ref, p_ref, y_ref, w1_ref, b1_ref, w2_ref, y2_ref):
  deg = degp_ref[0] + degp_ref[1] + 1.0
  dis = lax.rsqrt(deg)
  agg1 = dis * (p_ref[0] + p_ref[1] + y_ref[...])
  h1 = jnp.maximum(
      jnp.dot(agg1, w1_ref[...], preferred_element_type=jnp.float32)
      + b1_ref[...], 0.0)
  y2_ref[...] = dis * jnp.dot(h1, w2_ref[...],
                              preferred_element_type=jnp.float32)


def _tc_out(degp_ref, p_ref, y2_ref, b2_ref, o_ref):
  deg = degp_ref[0] + degp_ref[1] + 1.0
  dis = lax.rsqrt(deg)
  agg2 = dis * (p_ref[0] + p_ref[1] + y2_ref[...]) + b2_ref[...]
  m = jnp.max(agg2, axis=-1, keepdims=True)
  e = jnp.exp(agg2 - m)
  o_ref[...] = e / jnp.sum(e, axis=-1, keepdims=True)


def kernel(x, edge_index, W1, b1, W2, b2):
  n, d_in = x.shape
  e = edge_index.shape[1]
  d_hid = W1.shape[1]
  n_cls = W2.shape[1]

  ei32 = edge_index.astype(jnp.int32)
  ei3d = ei32.reshape(2, e // CH, CH)
  ei3s = ei32.reshape(2, e // CH_S, CH_S)

  ones_ch = jnp.ones((CH_S,), jnp.float32)
  zeros_n = jnp.zeros((n,), jnp.float32)
  zeros_1 = jnp.zeros((n // NS, d_in), jnp.float32)
  zeros_2 = jnp.zeros((n // NS, n_cls), jnp.float32)

  degp = _make_deg_kernel(n, e, CH_S)(ei3s, ones_ch, zeros_n)
  degp3 = degp.reshape(NC, n, 1)

  blk = 2000
  grid = (n // blk,)

  y = pl.pallas_call(
      _tc_prep,
      grid=grid,
      in_specs=[
          pl.BlockSpec((NC, blk, 1), lambda j: (0, j, 0)),
          pl.BlockSpec((blk, d_in), lambda j: (j, 0)),
      ],
      out_specs=pl.BlockSpec((blk, d_in), lambda j: (j, 0)),
      out_shape=jax.ShapeDtypeStruct((n, d_in), jnp.float32),
  )(degp3, x)

  part1 = _make_agg_kernel(n, e, d_in, CH)(y, ei3d, zeros_1)

  y2 = pl.pallas_call(
      _tc_mid,
      grid=grid,
      in_specs=[
          pl.BlockSpec((NC, blk, 1), lambda j: (0, j, 0)),
          pl.BlockSpec((NC, blk, d_in), lambda j: (0, j, 0)),
          pl.BlockSpec((blk, d_in), lambda j: (j, 0)),
          pl.BlockSpec((d_in, d_hid), lambda j: (0, 0)),
          pl.BlockSpec((1, d_hid), lambda j: (0, 0)),
          pl.BlockSpec((d_hid, n_cls), lambda j: (0, 0)),
      ],
      out_specs=pl.BlockSpec((blk, n_cls), lambda j: (j, 0)),
      out_shape=jax.ShapeDtypeStruct((n, n_cls), jnp.float32),
  )(degp3, part1, y, W1, b1.reshape(1, d_hid), W2)

  part2 = _make_agg_kernel(n, e, n_cls, CH_S, stage_src=True)(
      y2, ei3s, zeros_2)

  out = pl.pallas_call(
      _tc_out,
      grid=grid,
      in_specs=[
          pl.BlockSpec((NC, blk, 1), lambda j: (0, j, 0)),
          pl.BlockSpec((NC, blk, n_cls), lambda j: (0, j, 0)),
          pl.BlockSpec((blk, n_cls), lambda j: (j, 0)),
          pl.BlockSpec((1, n_cls), lambda j: (0, 0)),
      ],
      out_specs=pl.BlockSpec((blk, n_cls), lambda j: (j, 0)),
      out_shape=jax.ShapeDtypeStruct((n, n_cls), jnp.float32),
  )(degp3, part2, y2, b2.reshape(1, n_cls))

  return out
